# baseline (device time: 148515 ns/iter reference)
import functools

import jax
import jax.numpy as jnp
from jax import lax
from jax.experimental import pallas as pl
from jax.experimental.pallas import tpu as pltpu

N_DEV = 8
SQ = 1024
SKV = 1024
HQ = 8
DH = 128
CHUNK = SQ // N_DEV
SCALE = 0.08838834764831843


def kernel(x, Wq, K_ext, V_ext, Wo):
    def body(x_ref, wq_ref, k_ref, v_ref, wo_ref, out_ref,
             acc_ref, l_ref, comm_acc, comm_l,
             acc_ssem, acc_rsem, l_ssem, l_rsem, ag_ssem, ag_rsem):
        my = lax.axis_index("i")
        left = lax.rem(my - 1 + N_DEV, N_DEV)
        right = lax.rem(my + 1, N_DEV)

        barrier = pltpu.get_barrier_semaphore()
        for nbr in (left, right):
            pl.semaphore_signal(barrier, inc=1, device_id=(nbr,),
                                device_id_type=pl.DeviceIdType.MESH)
        pl.semaphore_wait(barrier, 2)

        xb = x_ref[0].astype(jnp.bfloat16)
        wqb = wq_ref[...].astype(jnp.bfloat16)
        q = lax.dot_general(xb, wqb, (((1,), (0,)), ((), ())),
                            preferred_element_type=jnp.float32)
        q = q.astype(jnp.bfloat16)

        qi = lax.broadcasted_iota(jnp.int32, (SQ, SKV), 0)
        ki = lax.broadcasted_iota(jnp.int32, (SQ, SKV), 1)
        mask = ((qi // 64) % 4) == ((ki // 64) % 4)

        for h in range(HQ):
            qh = q[:, h * DH:(h + 1) * DH]
            kh = k_ref[0, :, h, :].astype(jnp.bfloat16)
            s = lax.dot_general(qh, kh, (((1,), (1,)), ((), ())),
                                preferred_element_type=jnp.float32)
            w = jnp.where(mask, jnp.exp(s * SCALE), 0.0)
            l_ref[:, h] = jnp.sum(w, axis=1)
            vh = v_ref[0, :, h, :].astype(jnp.bfloat16)
            acc_ref[:, h, :] = lax.dot_general(
                w.astype(jnp.bfloat16), vh, (((1,), (0,)), ((), ())),
                preferred_element_type=jnp.float32)

        for t in range(N_DEV - 1):
            sc = lax.rem(my - 1 - t + N_DEV, N_DEV)
            if t == 0:
                src_acc = acc_ref.at[pl.ds(sc * CHUNK, CHUNK)]
                src_l = l_ref.at[pl.ds(sc * CHUNK, CHUNK)]
            else:
                src_acc = comm_acc.at[t - 1]
                src_l = comm_l.at[t - 1]
            rdma_a = pltpu.make_async_remote_copy(
                src_ref=src_acc, dst_ref=comm_acc.at[t],
                send_sem=acc_ssem.at[t], recv_sem=acc_rsem.at[t],
                device_id=(right,), device_id_type=pl.DeviceIdType.MESH)
            rdma_l = pltpu.make_async_remote_copy(
                src_ref=src_l, dst_ref=comm_l.at[t],
                send_sem=l_ssem.at[t], recv_sem=l_rsem.at[t],
                device_id=(right,), device_id_type=pl.DeviceIdType.MESH)
            rdma_a.start()
            rdma_l.start()
            rdma_a.wait()
            rdma_l.wait()
            rc = lax.rem(my - 2 - t + N_DEV, N_DEV)
            comm_acc[t] = comm_acc[t] + acc_ref[pl.ds(rc * CHUNK, CHUNK)]
            comm_l[t] = comm_l[t] + l_ref[pl.ds(rc * CHUNK, CHUNK)]

        ctx = comm_acc[N_DEV - 2] / comm_l[N_DEV - 2][:, :, None]
        ctxb = ctx.astype(jnp.bfloat16).reshape(CHUNK, HQ * DH)
        wob = wo_ref[...].astype(jnp.bfloat16)
        out_chunk = lax.dot_general(ctxb, wob, (((1,), (0,)), ((), ())),
                                    preferred_element_type=jnp.float32)
        out_ref[0, pl.ds(my * CHUNK, CHUNK), :] = out_chunk

        for g in range(N_DEV - 1):
            c = lax.rem(my - g + N_DEV, N_DEV)
            sl = pl.ds(c * CHUNK, CHUNK)
            rdma = pltpu.make_async_remote_copy(
                src_ref=out_ref.at[0, sl], dst_ref=out_ref.at[0, sl],
                send_sem=ag_ssem.at[g], recv_sem=ag_rsem.at[g],
                device_id=(right,), device_id_type=pl.DeviceIdType.MESH)
            rdma.start()
            rdma.wait()

        @functools.partial(pl.run_scoped,
                           second_barrier=pltpu.SemaphoreType.REGULAR)
        def _(second_barrier):
            for nbr in (left, right):
                pl.semaphore_signal(second_barrier, inc=1, device_id=(nbr,),
                                    device_id_type=pl.DeviceIdType.MESH)
            pl.semaphore_wait(second_barrier, 2)

    return pl.pallas_call(
        body,
        out_shape=jax.ShapeDtypeStruct((1, SQ, HQ * DH), jnp.float32),
        in_specs=[pl.BlockSpec(memory_space=pltpu.VMEM)] * 5,
        out_specs=pl.BlockSpec(memory_space=pltpu.VMEM),
        scratch_shapes=[
            pltpu.VMEM((SQ, HQ, DH), jnp.float32),
            pltpu.VMEM((SQ, HQ), jnp.float32),
            pltpu.VMEM((N_DEV - 1, CHUNK, HQ, DH), jnp.float32),
            pltpu.VMEM((N_DEV - 1, CHUNK, HQ), jnp.float32),
            pltpu.SemaphoreType.DMA((N_DEV - 1,)),
            pltpu.SemaphoreType.DMA((N_DEV - 1,)),
            pltpu.SemaphoreType.DMA((N_DEV - 1,)),
            pltpu.SemaphoreType.DMA((N_DEV - 1,)),
            pltpu.SemaphoreType.DMA((N_DEV - 1,)),
            pltpu.SemaphoreType.DMA((N_DEV - 1,)),
        ],
        compiler_params=pltpu.CompilerParams(collective_id=0),
    )(x, Wq, K_ext, V_ext, Wo)


# device time: 74565 ns/iter; 1.9918x vs baseline; 1.9918x over previous
import functools

import jax
import jax.numpy as jnp
from jax import lax
from jax.experimental import pallas as pl
from jax.experimental.pallas import tpu as pltpu

N_DEV = 8
SQ = 1024
SKV = 1024
HQ = 8
DH = 128
CHUNK = SQ // N_DEV
SCALE = 0.08838834764831843


def kernel(x, Wq, K_ext, V_ext, Wo):
    def body(x_ref, wq_ref, k_ref, v_ref, wo_ref, out_ref,
             acc_ref, l_ref, acc_bf, l_bf, rs_acc_slots, rs_l_slots,
             ag_send, ag_slots,
             acc_ssem, acc_rsem, l_ssem, l_rsem, ag_ssem, ag_rsem):
        my = lax.axis_index("i")

        barrier = pltpu.get_barrier_semaphore()
        for o in range(1, N_DEV):
            peer = lax.rem(my + o, N_DEV)
            pl.semaphore_signal(barrier, inc=1, device_id=(peer,),
                                device_id_type=pl.DeviceIdType.MESH)
        pl.semaphore_wait(barrier, N_DEV - 1)

        xb = x_ref[0].astype(jnp.bfloat16)
        wqb = wq_ref[...].astype(jnp.bfloat16)
        q = lax.dot_general(xb, wqb, (((1,), (0,)), ((), ())),
                            preferred_element_type=jnp.float32)
        q = q.astype(jnp.bfloat16)

        qi = lax.broadcasted_iota(jnp.int32, (SQ, SKV), 0)
        ki = lax.broadcasted_iota(jnp.int32, (SQ, SKV), 1)
        mask = ((qi // 64) % 4) == ((ki // 64) % 4)

        for h in range(HQ):
            qh = q[:, h * DH:(h + 1) * DH]
            kh = k_ref[0, :, h, :].astype(jnp.bfloat16)
            s = lax.dot_general(qh, kh, (((1,), (1,)), ((), ())),
                                preferred_element_type=jnp.float32)
            w = jnp.where(mask, jnp.exp(s * SCALE), 0.0)
            l_ref[:, h] = jnp.sum(w, axis=1)
            vh = v_ref[0, :, h, :].astype(jnp.bfloat16)
            acc_ref[:, h, :] = lax.dot_general(
                w.astype(jnp.bfloat16), vh, (((1,), (0,)), ((), ())),
                preferred_element_type=jnp.float32)

        acc_bf[...] = acc_ref[...].astype(jnp.bfloat16)
        l_bf[...] = l_ref[...].astype(jnp.bfloat16)

        rs_rdmas = []
        for o in range(1, N_DEV):
            peer = lax.rem(my + o, N_DEV)
            j = N_DEV - 1 - o
            sl = pl.ds(peer * CHUNK, CHUNK)
            rdma_a = pltpu.make_async_remote_copy(
                src_ref=acc_bf.at[sl], dst_ref=rs_acc_slots.at[j],
                send_sem=acc_ssem.at[j], recv_sem=acc_rsem.at[j],
                device_id=(peer,), device_id_type=pl.DeviceIdType.MESH)
            rdma_l = pltpu.make_async_remote_copy(
                src_ref=l_bf.at[sl], dst_ref=rs_l_slots.at[j],
                send_sem=l_ssem.at[j], recv_sem=l_rsem.at[j],
                device_id=(peer,), device_id_type=pl.DeviceIdType.MESH)
            rdma_a.start()
            rdma_l.start()
            rs_rdmas.append((rdma_a, rdma_l))

        for ra, rl in rs_rdmas:
            ra.wait_recv()
            rl.wait_recv()

        myl = pl.ds(my * CHUNK, CHUNK)
        tot_acc = acc_ref[myl]
        tot_l = l_ref[myl]
        for j in range(N_DEV - 1):
            tot_acc = tot_acc + rs_acc_slots[j].astype(jnp.float32)
            tot_l = tot_l + rs_l_slots[j].astype(jnp.float32)

        ctx = tot_acc / tot_l[:, :, None]
        ctxb = ctx.astype(jnp.bfloat16).reshape(CHUNK, HQ * DH)
        wob = wo_ref[...].astype(jnp.bfloat16)
        out_chunk = lax.dot_general(ctxb, wob, (((1,), (0,)), ((), ())),
                                    preferred_element_type=jnp.float32)
        out_ref[0, myl, :] = out_chunk
        ag_send[...] = out_chunk.astype(jnp.bfloat16)

        ag_rdmas = []
        for o in range(1, N_DEV):
            peer = lax.rem(my + o, N_DEV)
            j = N_DEV - 1 - o
            rdma = pltpu.make_async_remote_copy(
                src_ref=ag_send, dst_ref=ag_slots.at[j],
                send_sem=ag_ssem.at[j], recv_sem=ag_rsem.at[j],
                device_id=(peer,), device_id_type=pl.DeviceIdType.MESH)
            rdma.start()
            ag_rdmas.append(rdma)

        for j in range(N_DEV - 1):
            ag_rdmas[j].wait_recv()

        for j in range(N_DEV - 1):
            src = lax.rem(my + 1 + j, N_DEV)
            out_ref[0, pl.ds(src * CHUNK, CHUNK), :] = (
                ag_slots[j].astype(jnp.float32))

        for ra, rl in rs_rdmas:
            ra.wait_send()
            rl.wait_send()
        for rdma in ag_rdmas:
            rdma.wait_send()

        @functools.partial(pl.run_scoped,
                           second_barrier=pltpu.SemaphoreType.REGULAR)
        def _(second_barrier):
            for o in range(1, N_DEV):
                peer = lax.rem(my + o, N_DEV)
                pl.semaphore_signal(second_barrier, inc=1, device_id=(peer,),
                                    device_id_type=pl.DeviceIdType.MESH)
            pl.semaphore_wait(second_barrier, N_DEV - 1)

    return pl.pallas_call(
        body,
        out_shape=jax.ShapeDtypeStruct((1, SQ, HQ * DH), jnp.float32),
        in_specs=[pl.BlockSpec(memory_space=pltpu.VMEM)] * 5,
        out_specs=pl.BlockSpec(memory_space=pltpu.VMEM),
        scratch_shapes=[
            pltpu.VMEM((SQ, HQ, DH), jnp.float32),
            pltpu.VMEM((SQ, HQ), jnp.float32),
            pltpu.VMEM((SQ, HQ, DH), jnp.bfloat16),
            pltpu.VMEM((SQ, HQ), jnp.bfloat16),
            pltpu.VMEM((N_DEV - 1, CHUNK, HQ, DH), jnp.bfloat16),
            pltpu.VMEM((N_DEV - 1, CHUNK, HQ), jnp.bfloat16),
            pltpu.VMEM((CHUNK, HQ * DH), jnp.bfloat16),
            pltpu.VMEM((N_DEV - 1, CHUNK, HQ * DH), jnp.bfloat16),
            pltpu.SemaphoreType.DMA((N_DEV - 1,)),
            pltpu.SemaphoreType.DMA((N_DEV - 1,)),
            pltpu.SemaphoreType.DMA((N_DEV - 1,)),
            pltpu.SemaphoreType.DMA((N_DEV - 1,)),
            pltpu.SemaphoreType.DMA((N_DEV - 1,)),
            pltpu.SemaphoreType.DMA((N_DEV - 1,)),
        ],
        compiler_params=pltpu.CompilerParams(collective_id=0),
    )(x, Wq, K_ext, V_ext, Wo)


# device time: 63932 ns/iter; 2.3230x vs baseline; 1.1663x over previous
import functools

import jax
import jax.numpy as jnp
from jax import lax
from jax.experimental import pallas as pl
from jax.experimental.pallas import tpu as pltpu

N_DEV = 8
SQ = 1024
SKV = 1024
HQ = 8
DH = 128
CHUNK = SQ // N_DEV
NRES = 4
GRP = SQ // NRES
SCALE = 0.08838834764831843


def kernel(x, Wq, K_ext, V_ext, Wo):
    def body(x_ref, wq_ref, k_ref, v_ref, wo_ref, out_ref,
             acc_ref, l_ref, acc_bf, l_bf, rs_acc_slots, rs_l_slots,
             ag_send, ag_slots,
             acc_ssem, acc_rsem, l_ssem, l_rsem, ag_ssem, ag_rsem):
        my = lax.axis_index("i")

        barrier = pltpu.get_barrier_semaphore()
        for o in range(1, N_DEV):
            peer = lax.rem(my + o, N_DEV)
            pl.semaphore_signal(barrier, inc=1, device_id=(peer,),
                                device_id_type=pl.DeviceIdType.MESH)
        pl.semaphore_wait(barrier, N_DEV - 1)

        xb = x_ref[0].astype(jnp.bfloat16)
        wqb = wq_ref[...].astype(jnp.bfloat16)
        q = lax.dot_general(xb, wqb, (((1,), (0,)), ((), ())),
                            preferred_element_type=jnp.float32)
        q = (q * SCALE).astype(jnp.bfloat16)

        qp = q.reshape(NRES, NRES, 64, HQ * DH).transpose(1, 0, 2, 3)
        qp = qp.reshape(NRES, GRP, HQ * DH)
        kp = k_ref[0].astype(jnp.bfloat16).reshape(
            NRES, NRES, 64, HQ, DH).transpose(1, 0, 2, 3, 4)
        kp = kp.reshape(NRES, GRP, HQ, DH)
        vp = v_ref[0].astype(jnp.bfloat16).reshape(
            NRES, NRES, 64, HQ, DH).transpose(1, 0, 2, 3, 4)
        vp = vp.reshape(NRES, GRP, HQ, DH)

        def rs_descriptors(c):
            j = lax.rem(c - my - 1 + N_DEV, N_DEV)
            sl = pl.ds(c * CHUNK, CHUNK)
            rdma_a = pltpu.make_async_remote_copy(
                src_ref=acc_bf.at[sl], dst_ref=rs_acc_slots.at[j],
                send_sem=acc_ssem.at[j], recv_sem=acc_rsem.at[j],
                device_id=(c,), device_id_type=pl.DeviceIdType.MESH)
            rdma_l = pltpu.make_async_remote_copy(
                src_ref=l_bf.at[sl], dst_ref=rs_l_slots.at[j],
                send_sem=l_ssem.at[j], recv_sem=l_rsem.at[j],
                device_id=(c,), device_id_type=pl.DeviceIdType.MESH)
            return rdma_a, rdma_l

        for r in range(NRES):
            rrows = pl.ds(r * GRP, GRP)
            for h in range(HQ):
                qrh = qp[r, :, h * DH:(h + 1) * DH]
                s = lax.dot_general(qrh, kp[r, :, h, :],
                                    (((1,), (1,)), ((), ())),
                                    preferred_element_type=jnp.float32)
                w = jnp.exp(s)
                l_ref[rrows, h] = jnp.sum(w, axis=1)
                acc_ref[rrows, h, :] = lax.dot_general(
                    w.astype(jnp.bfloat16), vp[r, :, h, :],
                    (((1,), (0,)), ((), ())),
                    preferred_element_type=jnp.float32)
            acc_bf[rrows] = acc_ref[rrows].astype(jnp.bfloat16)
            l_bf[rrows] = l_ref[rrows].astype(jnp.bfloat16)
            for c in (2 * r, 2 * r + 1):
                @pl.when(my != c)
                def _(c=c):
                    rdma_a, rdma_l = rs_descriptors(c)
                    rdma_a.start()
                    rdma_l.start()

        for j in range(N_DEV - 1):
            pltpu.make_async_remote_copy(
                src_ref=rs_acc_slots.at[j], dst_ref=rs_acc_slots.at[j],
                send_sem=acc_ssem.at[j], recv_sem=acc_rsem.at[j],
                device_id=(my,),
                device_id_type=pl.DeviceIdType.MESH).wait_recv()
            pltpu.make_async_remote_copy(
                src_ref=rs_l_slots.at[j], dst_ref=rs_l_slots.at[j],
                send_sem=l_ssem.at[j], recv_sem=l_rsem.at[j],
                device_id=(my,),
                device_id_type=pl.DeviceIdType.MESH).wait_recv()

        myl = pl.ds(my * CHUNK, CHUNK)
        tot_acc = acc_ref[myl]
        tot_l = l_ref[myl]
        for j in range(N_DEV - 1):
            tot_acc = tot_acc + rs_acc_slots[j].astype(jnp.float32)
            tot_l = tot_l + rs_l_slots[j].astype(jnp.float32)

        ctx = tot_acc / tot_l[:, :, None]
        ctxb = ctx.astype(jnp.bfloat16).reshape(CHUNK, HQ * DH)
        wob = wo_ref[...].astype(jnp.bfloat16)
        out_chunk = lax.dot_general(ctxb, wob, (((1,), (0,)), ((), ())),
                                    preferred_element_type=jnp.float32)
        ag_send[...] = out_chunk.astype(jnp.bfloat16)

        def store_chunk(c, chunk_f32):
            b0 = (8 * lax.rem(c, 2) + lax.div(c, 2)) * 64
            out_ref[0, pl.ds(b0, 64), :] = chunk_f32[:64]
            out_ref[0, pl.ds(b0 + 256, 64), :] = chunk_f32[64:]

        store_chunk(my, out_chunk)

        ag_rdmas = []
        for o in range(1, N_DEV):
            peer = lax.rem(my + o, N_DEV)
            j = N_DEV - 1 - o
            rdma = pltpu.make_async_remote_copy(
                src_ref=ag_send, dst_ref=ag_slots.at[j],
                send_sem=ag_ssem.at[j], recv_sem=ag_rsem.at[j],
                device_id=(peer,), device_id_type=pl.DeviceIdType.MESH)
            rdma.start()
            ag_rdmas.append(rdma)

        for j in range(N_DEV - 1):
            pltpu.make_async_remote_copy(
                src_ref=ag_slots.at[j], dst_ref=ag_slots.at[j],
                send_sem=ag_ssem.at[j], recv_sem=ag_rsem.at[j],
                device_id=(my,),
                device_id_type=pl.DeviceIdType.MESH).wait_recv()
            src = lax.rem(my + 1 + j, N_DEV)
            store_chunk(src, ag_slots[j].astype(jnp.float32))

        for c in range(N_DEV):
            @pl.when(my != c)
            def _(c=c):
                rdma_a, rdma_l = rs_descriptors(c)
                rdma_a.wait_send()
                rdma_l.wait_send()
        for rdma in ag_rdmas:
            rdma.wait_send()

        @functools.partial(pl.run_scoped,
                           second_barrier=pltpu.SemaphoreType.REGULAR)
        def _(second_barrier):
            for o in range(1, N_DEV):
                peer = lax.rem(my + o, N_DEV)
                pl.semaphore_signal(second_barrier, inc=1, device_id=(peer,),
                                    device_id_type=pl.DeviceIdType.MESH)
            pl.semaphore_wait(second_barrier, N_DEV - 1)

    return pl.pallas_call(
        body,
        out_shape=jax.ShapeDtypeStruct((1, SQ, HQ * DH), jnp.float32),
        in_specs=[pl.BlockSpec(memory_space=pltpu.VMEM)] * 5,
        out_specs=pl.BlockSpec(memory_space=pltpu.VMEM),
        scratch_shapes=[
            pltpu.VMEM((SQ, HQ, DH), jnp.float32),
            pltpu.VMEM((SQ, HQ), jnp.float32),
            pltpu.VMEM((SQ, HQ, DH), jnp.bfloat16),
            pltpu.VMEM((SQ, HQ), jnp.bfloat16),
            pltpu.VMEM((N_DEV - 1, CHUNK, HQ, DH), jnp.bfloat16),
            pltpu.VMEM((N_DEV - 1, CHUNK, HQ), jnp.bfloat16),
            pltpu.VMEM((CHUNK, HQ * DH), jnp.bfloat16),
            pltpu.VMEM((N_DEV - 1, CHUNK, HQ * DH), jnp.bfloat16),
            pltpu.SemaphoreType.DMA((N_DEV - 1,)),
            pltpu.SemaphoreType.DMA((N_DEV - 1,)),
            pltpu.SemaphoreType.DMA((N_DEV - 1,)),
            pltpu.SemaphoreType.DMA((N_DEV - 1,)),
            pltpu.SemaphoreType.DMA((N_DEV - 1,)),
            pltpu.SemaphoreType.DMA((N_DEV - 1,)),
        ],
        compiler_params=pltpu.CompilerParams(collective_id=0),
    )(x, Wq, K_ext, V_ext, Wo)


# device time: 55959 ns/iter; 2.6540x vs baseline; 1.1425x over previous
import functools
import os

import jax
import jax.numpy as jnp
from jax import lax
from jax.experimental import pallas as pl
from jax.experimental.pallas import tpu as pltpu

N_DEV = 8
SQ = 1024
SKV = 1024
HQ = 8
DH = 128
CHUNK = SQ // N_DEV
NRES = 4
GRP = SQ // NRES
SCALE = 0.08838834764831843

_SKIP_RS = os.environ.get("K_SKIP_RS") == "1"
_SKIP_AG = os.environ.get("K_SKIP_AG") == "1"


def kernel(x, Wq, K_ext, V_ext, Wo):
    def body(x_ref, wq_ref, k_ref, v_ref, wo_ref, out_ref,
             acc_ref, l_ref, acc_bf, l_bf, rs_acc_slots, rs_l_slots,
             ag_send, ag_slots,
             acc_ssem, acc_rsem, l_ssem, l_rsem, ag_ssem, ag_rsem):
        my = lax.axis_index("i")

        if not (_SKIP_RS and _SKIP_AG):
            barrier = pltpu.get_barrier_semaphore()
            for o in range(1, N_DEV):
                peer = lax.rem(my + o, N_DEV)
                pl.semaphore_signal(barrier, inc=1, device_id=(peer,),
                                    device_id_type=pl.DeviceIdType.MESH)
            pl.semaphore_wait(barrier, N_DEV - 1)

        xb = x_ref[0].astype(jnp.bfloat16)
        wqb = wq_ref[...].astype(jnp.bfloat16)
        q = lax.dot_general(xb, wqb, (((1,), (0,)), ((), ())),
                            preferred_element_type=jnp.float32)
        q = (q * SCALE).astype(jnp.bfloat16)

        qp = q.reshape(NRES, NRES, 64, HQ * DH).transpose(1, 0, 2, 3)
        qp = qp.reshape(NRES, GRP, HQ * DH)
        kp = k_ref[0].astype(jnp.bfloat16).reshape(
            NRES, NRES, 64, HQ, DH).transpose(1, 0, 2, 3, 4)
        kp = kp.reshape(NRES, GRP, HQ, DH)
        vp = v_ref[0].astype(jnp.bfloat16).reshape(
            NRES, NRES, 64, HQ, DH).transpose(1, 0, 2, 3, 4)
        vp = vp.reshape(NRES, GRP, HQ, DH)

        def rs_descriptor(c, h):
            j = lax.rem(c - my - 1 + N_DEV, N_DEV)
            return pltpu.make_async_remote_copy(
                src_ref=acc_bf.at[h, pl.ds(c * CHUNK, CHUNK), :],
                dst_ref=rs_acc_slots.at[h, j],
                send_sem=acc_ssem.at[h, j], recv_sem=acc_rsem.at[h, j],
                device_id=(c,), device_id_type=pl.DeviceIdType.MESH)

        def rs_l_descriptor(c):
            j = lax.rem(c - my - 1 + N_DEV, N_DEV)
            return pltpu.make_async_remote_copy(
                src_ref=l_bf.at[pl.ds(c * CHUNK, CHUNK)],
                dst_ref=rs_l_slots.at[j],
                send_sem=l_ssem.at[j], recv_sem=l_rsem.at[j],
                device_id=(c,), device_id_type=pl.DeviceIdType.MESH)

        for h in range(HQ):
            for r in range(NRES):
                rrows = pl.ds(r * GRP, GRP)
                qrh = qp[r, :, h * DH:(h + 1) * DH]
                s = lax.dot_general(qrh, kp[r, :, h, :],
                                    (((1,), (1,)), ((), ())),
                                    preferred_element_type=jnp.float32)
                w = jnp.exp(s)
                l_ref[rrows, h] = jnp.sum(w, axis=1)
                acc_ref[h, rrows, :] = lax.dot_general(
                    w.astype(jnp.bfloat16), vp[r, :, h, :],
                    (((1,), (0,)), ((), ())),
                    preferred_element_type=jnp.float32)
            acc_bf[h] = acc_ref[h].astype(jnp.bfloat16)
            if not _SKIP_RS:
                for c in range(N_DEV):
                    @pl.when(my != c)
                    def _(c=c, h=h):
                        rs_descriptor(c, h).start()

        l_bf[...] = l_ref[...].astype(jnp.bfloat16)
        if not _SKIP_RS:
            for c in range(N_DEV):
                @pl.when(my != c)
                def _(c=c):
                    rs_l_descriptor(c).start()

        myl = pl.ds(my * CHUNK, CHUNK)
        if not _SKIP_RS:
            for h in range(HQ):
                for j in range(N_DEV - 1):
                    pltpu.make_async_remote_copy(
                        src_ref=rs_acc_slots.at[h, j],
                        dst_ref=rs_acc_slots.at[h, j],
                        send_sem=acc_ssem.at[h, j], recv_sem=acc_rsem.at[h, j],
                        device_id=(my,),
                        device_id_type=pl.DeviceIdType.MESH).wait_recv()
            for j in range(N_DEV - 1):
                pltpu.make_async_remote_copy(
                    src_ref=rs_l_slots.at[j], dst_ref=rs_l_slots.at[j],
                    send_sem=l_ssem.at[j], recv_sem=l_rsem.at[j],
                    device_id=(my,),
                    device_id_type=pl.DeviceIdType.MESH).wait_recv()

        tot_l = l_ref[myl]
        if not _SKIP_RS:
            for j in range(N_DEV - 1):
                tot_l = tot_l + rs_l_slots[j].astype(jnp.float32)

        ctx_parts = []
        for h in range(HQ):
            tot_h = acc_ref[h, myl, :]
            if not _SKIP_RS:
                for j in range(N_DEV - 1):
                    tot_h = tot_h + rs_acc_slots[h, j].astype(jnp.float32)
            ctx_parts.append((tot_h / tot_l[:, h, None]).astype(jnp.bfloat16))
        ctxb = jnp.concatenate(ctx_parts, axis=1)

        wob = wo_ref[...].astype(jnp.bfloat16)
        out_chunk = lax.dot_general(ctxb, wob, (((1,), (0,)), ((), ())),
                                    preferred_element_type=jnp.float32)
        ag_send[...] = out_chunk.astype(jnp.bfloat16)

        def store_chunk(c, chunk_f32):
            b0 = (8 * lax.rem(c, 2) + lax.div(c, 2)) * 64
            out_ref[0, pl.ds(b0, 64), :] = chunk_f32[:64]
            out_ref[0, pl.ds(b0 + 256, 64), :] = chunk_f32[64:]

        store_chunk(my, out_chunk)

        ag_rdmas = []
        if not _SKIP_AG:
            for o in range(1, N_DEV):
                peer = lax.rem(my + o, N_DEV)
                j = N_DEV - 1 - o
                rdma = pltpu.make_async_remote_copy(
                    src_ref=ag_send, dst_ref=ag_slots.at[j],
                    send_sem=ag_ssem.at[j], recv_sem=ag_rsem.at[j],
                    device_id=(peer,), device_id_type=pl.DeviceIdType.MESH)
                rdma.start()
                ag_rdmas.append(rdma)

            for j in range(N_DEV - 1):
                pltpu.make_async_remote_copy(
                    src_ref=ag_slots.at[j], dst_ref=ag_slots.at[j],
                    send_sem=ag_ssem.at[j], recv_sem=ag_rsem.at[j],
                    device_id=(my,),
                    device_id_type=pl.DeviceIdType.MESH).wait_recv()
                src = lax.rem(my + 1 + j, N_DEV)
                store_chunk(src, ag_slots[j].astype(jnp.float32))

        if not _SKIP_RS:
            for c in range(N_DEV):
                @pl.when(my != c)
                def _(c=c):
                    for h in range(HQ):
                        rs_descriptor(c, h).wait_send()
                    rs_l_descriptor(c).wait_send()
        for rdma in ag_rdmas:
            rdma.wait_send()

        if not (_SKIP_RS and _SKIP_AG):
            @functools.partial(pl.run_scoped,
                               second_barrier=pltpu.SemaphoreType.REGULAR)
            def _(second_barrier):
                for o in range(1, N_DEV):
                    peer = lax.rem(my + o, N_DEV)
                    pl.semaphore_signal(second_barrier, inc=1,
                                        device_id=(peer,),
                                        device_id_type=pl.DeviceIdType.MESH)
                pl.semaphore_wait(second_barrier, N_DEV - 1)

    return pl.pallas_call(
        body,
        out_shape=jax.ShapeDtypeStruct((1, SQ, HQ * DH), jnp.float32),
        in_specs=[pl.BlockSpec(memory_space=pltpu.VMEM)] * 5,
        out_specs=pl.BlockSpec(memory_space=pltpu.VMEM),
        scratch_shapes=[
            pltpu.VMEM((HQ, SQ, DH), jnp.float32),
            pltpu.VMEM((SQ, HQ), jnp.float32),
            pltpu.VMEM((HQ, SQ, DH), jnp.bfloat16),
            pltpu.VMEM((SQ, HQ), jnp.bfloat16),
            pltpu.VMEM((HQ, N_DEV - 1, CHUNK, DH), jnp.bfloat16),
            pltpu.VMEM((N_DEV - 1, CHUNK, HQ), jnp.bfloat16),
            pltpu.VMEM((CHUNK, HQ * DH), jnp.bfloat16),
            pltpu.VMEM((N_DEV - 1, CHUNK, HQ * DH), jnp.bfloat16),
            pltpu.SemaphoreType.DMA((HQ, N_DEV - 1)),
            pltpu.SemaphoreType.DMA((HQ, N_DEV - 1)),
            pltpu.SemaphoreType.DMA((N_DEV - 1,)),
            pltpu.SemaphoreType.DMA((N_DEV - 1,)),
            pltpu.SemaphoreType.DMA((N_DEV - 1,)),
            pltpu.SemaphoreType.DMA((N_DEV - 1,)),
        ],
        compiler_params=(None if (_SKIP_RS and _SKIP_AG)
                         else pltpu.CompilerParams(collective_id=0)),
    )(x, Wq, K_ext, V_ext, Wo)


# device time: 53206 ns/iter; 2.7913x vs baseline; 1.0517x over previous
import functools
import os

import jax
import jax.numpy as jnp
from jax import lax
from jax.experimental import pallas as pl
from jax.experimental.pallas import tpu as pltpu

N_DEV = 8
SQ = 1024
SKV = 1024
HQ = 8
DH = 128
CHUNK = SQ // N_DEV
NRES = 4
GRP = SQ // NRES
SCALE = 0.08838834764831843

_SKIP_RS = os.environ.get("K_SKIP_RS") == "1"
_SKIP_AG = os.environ.get("K_SKIP_AG") == "1"


def kernel(x, Wq, K_ext, V_ext, Wo):
    def body(x_ref, wq_ref, k_ref, v_ref, wo_ref, out_ref,
             acc_ref, l_ref, acc_bf, l_bf, rs_acc_slots, rs_l_slots,
             ag_send, ag_slots, ag_sc_send, ag_sc_slots,
             acc_ssem, acc_rsem, l_ssem, l_rsem, ag_ssem, ag_rsem,
             sc_ssem, sc_rsem):
        my = lax.axis_index("i")

        if not (_SKIP_RS and _SKIP_AG):
            barrier = pltpu.get_barrier_semaphore()
            for o in range(1, N_DEV):
                peer = lax.rem(my + o, N_DEV)
                pl.semaphore_signal(barrier, inc=1, device_id=(peer,),
                                    device_id_type=pl.DeviceIdType.MESH)
            pl.semaphore_wait(barrier, N_DEV - 1)

        xb = x_ref[0].astype(jnp.bfloat16)
        wqb = wq_ref[...].astype(jnp.bfloat16)
        q = lax.dot_general(xb, wqb, (((1,), (0,)), ((), ())),
                            preferred_element_type=jnp.float32)
        q = (q * SCALE).astype(jnp.bfloat16)

        qp = q.reshape(NRES, NRES, 64, HQ * DH).transpose(1, 0, 2, 3)
        qp = qp.reshape(NRES, GRP, HQ * DH)
        kp = k_ref[0].astype(jnp.bfloat16).reshape(
            NRES, NRES, 64, HQ, DH).transpose(1, 0, 2, 3, 4)
        kp = kp.reshape(NRES, GRP, HQ, DH)
        vp = v_ref[0].astype(jnp.bfloat16).reshape(
            NRES, NRES, 64, HQ, DH).transpose(1, 0, 2, 3, 4)
        vp = vp.reshape(NRES, GRP, HQ, DH)

        def rs_descriptor(c, hp):
            j = lax.rem(c - my - 1 + N_DEV, N_DEV)
            return pltpu.make_async_remote_copy(
                src_ref=acc_bf.at[2 * hp:2 * hp + 2, pl.ds(c * CHUNK, CHUNK), :],
                dst_ref=rs_acc_slots.at[hp, j],
                send_sem=acc_ssem.at[hp, j], recv_sem=acc_rsem.at[hp, j],
                device_id=(c,), device_id_type=pl.DeviceIdType.MESH)

        def rs_l_descriptor(c):
            j = lax.rem(c - my - 1 + N_DEV, N_DEV)
            return pltpu.make_async_remote_copy(
                src_ref=l_bf.at[pl.ds(c * CHUNK, CHUNK)],
                dst_ref=rs_l_slots.at[j],
                send_sem=l_ssem.at[j], recv_sem=l_rsem.at[j],
                device_id=(c,), device_id_type=pl.DeviceIdType.MESH)

        for h in range(HQ):
            for r in range(NRES):
                rrows = pl.ds(r * GRP, GRP)
                qrh = qp[r, :, h * DH:(h + 1) * DH]
                s = lax.dot_general(qrh, kp[r, :, h, :],
                                    (((1,), (1,)), ((), ())),
                                    preferred_element_type=jnp.float32)
                w = jnp.exp(s)
                l_ref[rrows, h] = jnp.sum(w, axis=1)
                acc_ref[h, rrows, :] = lax.dot_general(
                    w.astype(jnp.bfloat16), vp[r, :, h, :],
                    (((1,), (0,)), ((), ())),
                    preferred_element_type=jnp.float32)
            acc_bf[h] = acc_ref[h].astype(jnp.bfloat16)
            if not _SKIP_RS and h % 2 == 1:
                for c in range(N_DEV):
                    @pl.when(my != c)
                    def _(c=c, hp=h // 2):
                        rs_descriptor(c, hp).start()

        l_bf[...] = l_ref[...].astype(jnp.bfloat16)
        if not _SKIP_RS:
            for c in range(N_DEV):
                @pl.when(my != c)
                def _(c=c):
                    rs_l_descriptor(c).start()

        myl = pl.ds(my * CHUNK, CHUNK)
        if not _SKIP_RS:
            for hp in range(HQ // 2):
                for j in range(N_DEV - 1):
                    pltpu.make_async_remote_copy(
                        src_ref=rs_acc_slots.at[hp, j],
                        dst_ref=rs_acc_slots.at[hp, j],
                        send_sem=acc_ssem.at[hp, j], recv_sem=acc_rsem.at[hp, j],
                        device_id=(my,),
                        device_id_type=pl.DeviceIdType.MESH).wait_recv()
            for j in range(N_DEV - 1):
                pltpu.make_async_remote_copy(
                    src_ref=rs_l_slots.at[j], dst_ref=rs_l_slots.at[j],
                    send_sem=l_ssem.at[j], recv_sem=l_rsem.at[j],
                    device_id=(my,),
                    device_id_type=pl.DeviceIdType.MESH).wait_recv()

        tot_l = l_ref[myl]
        if not _SKIP_RS:
            for j in range(N_DEV - 1):
                tot_l = tot_l + rs_l_slots[j].astype(jnp.float32)

        ctx_parts = []
        for h in range(HQ):
            tot_h = acc_ref[h, myl, :]
            if not _SKIP_RS:
                for j in range(N_DEV - 1):
                    tot_h = tot_h + rs_acc_slots[h // 2, j, h % 2].astype(
                        jnp.float32)
            ctx_parts.append((tot_h / tot_l[:, h, None]).astype(jnp.bfloat16))
        ctxb = jnp.concatenate(ctx_parts, axis=1)

        wob = wo_ref[...].astype(jnp.bfloat16)
        out_chunk = lax.dot_general(ctxb, wob, (((1,), (0,)), ((), ())),
                                    preferred_element_type=jnp.float32)
        row_scale = (jnp.max(jnp.abs(out_chunk), axis=1, keepdims=True)
                     * (1.0 / 127.0) + 1e-20)
        ag_send[...] = jnp.round(out_chunk / row_scale).astype(jnp.int8)
        ag_sc_send[...] = row_scale

        def store_chunk(c, chunk_f32):
            b0 = (8 * lax.rem(c, 2) + lax.div(c, 2)) * 64
            out_ref[0, pl.ds(b0, 64), :] = chunk_f32[:64]
            out_ref[0, pl.ds(b0 + 256, 64), :] = chunk_f32[64:]

        store_chunk(my, out_chunk)

        ag_rdmas = []
        if not _SKIP_AG:
            for o in range(1, N_DEV):
                peer = lax.rem(my + o, N_DEV)
                j = N_DEV - 1 - o
                rdma = pltpu.make_async_remote_copy(
                    src_ref=ag_send, dst_ref=ag_slots.at[j],
                    send_sem=ag_ssem.at[j], recv_sem=ag_rsem.at[j],
                    device_id=(peer,), device_id_type=pl.DeviceIdType.MESH)
                rdma.start()
                rdma_sc = pltpu.make_async_remote_copy(
                    src_ref=ag_sc_send, dst_ref=ag_sc_slots.at[j],
                    send_sem=sc_ssem.at[j], recv_sem=sc_rsem.at[j],
                    device_id=(peer,), device_id_type=pl.DeviceIdType.MESH)
                rdma_sc.start()
                ag_rdmas.extend((rdma, rdma_sc))

            for j in range(N_DEV - 1):
                pltpu.make_async_remote_copy(
                    src_ref=ag_slots.at[j], dst_ref=ag_slots.at[j],
                    send_sem=ag_ssem.at[j], recv_sem=ag_rsem.at[j],
                    device_id=(my,),
                    device_id_type=pl.DeviceIdType.MESH).wait_recv()
                pltpu.make_async_remote_copy(
                    src_ref=ag_sc_slots.at[j], dst_ref=ag_sc_slots.at[j],
                    send_sem=sc_ssem.at[j], recv_sem=sc_rsem.at[j],
                    device_id=(my,),
                    device_id_type=pl.DeviceIdType.MESH).wait_recv()
                src = lax.rem(my + 1 + j, N_DEV)
                store_chunk(src, ag_slots[j].astype(jnp.float32)
                            * ag_sc_slots[j])

        if not _SKIP_RS:
            for c in range(N_DEV):
                @pl.when(my != c)
                def _(c=c):
                    for hp in range(HQ // 2):
                        rs_descriptor(c, hp).wait_send()
                    rs_l_descriptor(c).wait_send()
        for rdma in ag_rdmas:
            rdma.wait_send()

        if not (_SKIP_RS and _SKIP_AG):
            @functools.partial(pl.run_scoped,
                               second_barrier=pltpu.SemaphoreType.REGULAR)
            def _(second_barrier):
                for o in range(1, N_DEV):
                    peer = lax.rem(my + o, N_DEV)
                    pl.semaphore_signal(second_barrier, inc=1,
                                        device_id=(peer,),
                                        device_id_type=pl.DeviceIdType.MESH)
                pl.semaphore_wait(second_barrier, N_DEV - 1)

    return pl.pallas_call(
        body,
        out_shape=jax.ShapeDtypeStruct((1, SQ, HQ * DH), jnp.float32),
        in_specs=[pl.BlockSpec(memory_space=pltpu.VMEM)] * 5,
        out_specs=pl.BlockSpec(memory_space=pltpu.VMEM),
        scratch_shapes=[
            pltpu.VMEM((HQ, SQ, DH), jnp.float32),
            pltpu.VMEM((SQ, HQ), jnp.float32),
            pltpu.VMEM((HQ, SQ, DH), jnp.bfloat16),
            pltpu.VMEM((SQ, HQ), jnp.bfloat16),
            pltpu.VMEM((HQ // 2, N_DEV - 1, 2, CHUNK, DH),
                       jnp.bfloat16),
            pltpu.VMEM((N_DEV - 1, CHUNK, HQ), jnp.bfloat16),
            pltpu.VMEM((CHUNK, HQ * DH), jnp.int8),
            pltpu.VMEM((N_DEV - 1, CHUNK, HQ * DH), jnp.int8),
            pltpu.VMEM((CHUNK, 1), jnp.float32),
            pltpu.VMEM((N_DEV - 1, CHUNK, 1), jnp.float32),
            pltpu.SemaphoreType.DMA((HQ // 2, N_DEV - 1)),
            pltpu.SemaphoreType.DMA((HQ // 2, N_DEV - 1)),
            pltpu.SemaphoreType.DMA((N_DEV - 1,)),
            pltpu.SemaphoreType.DMA((N_DEV - 1,)),
            pltpu.SemaphoreType.DMA((N_DEV - 1,)),
            pltpu.SemaphoreType.DMA((N_DEV - 1,)),
            pltpu.SemaphoreType.DMA((N_DEV - 1,)),
            pltpu.SemaphoreType.DMA((N_DEV - 1,)),
        ],
        compiler_params=(None if (_SKIP_RS and _SKIP_AG)
                         else pltpu.CompilerParams(collective_id=0)),
    )(x, Wq, K_ext, V_ext, Wo)


# device time: 49077 ns/iter; 3.0262x vs baseline; 1.0841x over previous
import functools
import os

import jax
import jax.numpy as jnp
from jax import lax
from jax.experimental import pallas as pl
from jax.experimental.pallas import tpu as pltpu

N_DEV = 8
SQ = 1024
SKV = 1024
HQ = 8
DH = 128
CHUNK = SQ // N_DEV
NRES = 4
GRP = SQ // NRES
SCALE = 0.08838834764831843

_SKIP_RS = os.environ.get("K_SKIP_RS") == "1"
_SKIP_AG = os.environ.get("K_SKIP_AG") == "1"


def kernel(x, Wq, K_ext, V_ext, Wo):
    def body(x_ref, wq_ref, k_ref, v_ref, wo_ref, out_ref,
             acc_ref, lw_ref, acc_i8, rs_acc_slots, rs_lw_slots,
             ag_send, ag_slots, ag_sc_send, ag_sc_slots,
             acc_ssem, acc_rsem, l_ssem, l_rsem, ag_ssem, ag_rsem,
             sc_ssem, sc_rsem):
        my = lax.axis_index("i")

        if not (_SKIP_RS and _SKIP_AG):
            barrier = pltpu.get_barrier_semaphore()
            for o in range(1, N_DEV):
                peer = lax.rem(my + o, N_DEV)
                pl.semaphore_signal(barrier, inc=1, device_id=(peer,),
                                    device_id_type=pl.DeviceIdType.MESH)
            pl.semaphore_wait(barrier, N_DEV - 1)

        xb = x_ref[0].astype(jnp.bfloat16)
        wqb = wq_ref[...].astype(jnp.bfloat16)
        q = lax.dot_general(xb, wqb, (((1,), (0,)), ((), ())),
                            preferred_element_type=jnp.float32)
        q = (q * SCALE).astype(jnp.bfloat16)

        qp = q.reshape(NRES, NRES, 64, HQ * DH).transpose(1, 0, 2, 3)
        qp = qp.reshape(NRES, GRP, HQ * DH)
        kp = k_ref[0].astype(jnp.bfloat16).reshape(
            NRES, NRES, 64, HQ, DH).transpose(1, 0, 2, 3, 4)
        kp = kp.reshape(NRES, GRP, HQ, DH)
        vp = v_ref[0].astype(jnp.bfloat16).reshape(
            NRES, NRES, 64, HQ, DH).transpose(1, 0, 2, 3, 4)
        vp = vp.reshape(NRES, GRP, HQ, DH)

        def rs_descriptor(c, hp):
            j = lax.rem(c - my - 1 + N_DEV, N_DEV)
            return pltpu.make_async_remote_copy(
                src_ref=acc_i8.at[2 * hp:2 * hp + 2, pl.ds(c * CHUNK, CHUNK), :],
                dst_ref=rs_acc_slots.at[hp, j],
                send_sem=acc_ssem.at[hp, j], recv_sem=acc_rsem.at[hp, j],
                device_id=(c,), device_id_type=pl.DeviceIdType.MESH)

        def rs_lw_descriptor(c):
            j = lax.rem(c - my - 1 + N_DEV, N_DEV)
            return pltpu.make_async_remote_copy(
                src_ref=lw_ref.at[pl.ds(c * CHUNK, CHUNK)],
                dst_ref=rs_lw_slots.at[j],
                send_sem=l_ssem.at[j], recv_sem=l_rsem.at[j],
                device_id=(c,), device_id_type=pl.DeviceIdType.MESH)

        for h in range(HQ):
            for r in range(NRES):
                rrows = pl.ds(r * GRP, GRP)
                qrh = qp[r, :, h * DH:(h + 1) * DH]
                s = lax.dot_general(qrh, kp[r, :, h, :],
                                    (((1,), (1,)), ((), ())),
                                    preferred_element_type=jnp.float32)
                w = jnp.exp(s)
                lw_ref[rrows, h] = jnp.sum(w, axis=1)
                acc_ref[h, rrows, :] = lax.dot_general(
                    w.astype(jnp.bfloat16), vp[r, :, h, :],
                    (((1,), (0,)), ((), ())),
                    preferred_element_type=jnp.float32)
            acc_h = acc_ref[h]
            qsc = (jnp.max(jnp.abs(acc_h), axis=1, keepdims=True)
                   * (1.0 / 127.0) + 1e-20)
            acc_i8[h] = jnp.round(acc_h / qsc).astype(jnp.int8)
            lw_ref[:, 8 + h] = qsc[:, 0]
            if not _SKIP_RS and h % 2 == 1:
                for c in range(N_DEV):
                    @pl.when(my != c)
                    def _(c=c, hp=h // 2):
                        rs_descriptor(c, hp).start()

        if not _SKIP_RS:
            for c in range(N_DEV):
                @pl.when(my != c)
                def _(c=c):
                    rs_lw_descriptor(c).start()

        myl = pl.ds(my * CHUNK, CHUNK)
        if not _SKIP_RS:
            for hp in range(HQ // 2):
                for j in range(N_DEV - 1):
                    pltpu.make_async_remote_copy(
                        src_ref=rs_acc_slots.at[hp, j],
                        dst_ref=rs_acc_slots.at[hp, j],
                        send_sem=acc_ssem.at[hp, j], recv_sem=acc_rsem.at[hp, j],
                        device_id=(my,),
                        device_id_type=pl.DeviceIdType.MESH).wait_recv()
            for j in range(N_DEV - 1):
                pltpu.make_async_remote_copy(
                    src_ref=rs_lw_slots.at[j], dst_ref=rs_lw_slots.at[j],
                    send_sem=l_ssem.at[j], recv_sem=l_rsem.at[j],
                    device_id=(my,),
                    device_id_type=pl.DeviceIdType.MESH).wait_recv()

        tot_l = lw_ref[myl, 0:8]
        lw_in = []
        if not _SKIP_RS:
            lw_in = [rs_lw_slots[j] for j in range(N_DEV - 1)]
            for j in range(N_DEV - 1):
                tot_l = tot_l + lw_in[j][:, 0:8]

        ctx_parts = []
        for h in range(HQ):
            tot_h = acc_ref[h, myl, :]
            if not _SKIP_RS:
                for j in range(N_DEV - 1):
                    tot_h = tot_h + (
                        rs_acc_slots[h // 2, j, h % 2].astype(jnp.float32)
                        * lw_in[j][:, 8 + h, None])
            ctx_parts.append((tot_h / tot_l[:, h, None]).astype(jnp.bfloat16))
        ctxb = jnp.concatenate(ctx_parts, axis=1)

        wob = wo_ref[...].astype(jnp.bfloat16)
        out_chunk = lax.dot_general(ctxb, wob, (((1,), (0,)), ((), ())),
                                    preferred_element_type=jnp.float32)
        row_scale = (jnp.max(jnp.abs(out_chunk), axis=1, keepdims=True)
                     * (1.0 / 127.0) + 1e-20)
        ag_send[...] = jnp.round(out_chunk / row_scale).astype(jnp.int8)
        ag_sc_send[...] = row_scale

        def store_chunk(c, chunk_f32):
            b0 = (8 * lax.rem(c, 2) + lax.div(c, 2)) * 64
            out_ref[0, pl.ds(b0, 64), :] = chunk_f32[:64]
            out_ref[0, pl.ds(b0 + 256, 64), :] = chunk_f32[64:]

        store_chunk(my, out_chunk)

        ag_rdmas = []
        if not _SKIP_AG:
            for o in range(1, N_DEV):
                peer = lax.rem(my + o, N_DEV)
                j = N_DEV - 1 - o
                rdma = pltpu.make_async_remote_copy(
                    src_ref=ag_send, dst_ref=ag_slots.at[j],
                    send_sem=ag_ssem.at[j], recv_sem=ag_rsem.at[j],
                    device_id=(peer,), device_id_type=pl.DeviceIdType.MESH)
                rdma.start()
                rdma_sc = pltpu.make_async_remote_copy(
                    src_ref=ag_sc_send, dst_ref=ag_sc_slots.at[j],
                    send_sem=sc_ssem.at[j], recv_sem=sc_rsem.at[j],
                    device_id=(peer,), device_id_type=pl.DeviceIdType.MESH)
                rdma_sc.start()
                ag_rdmas.extend((rdma, rdma_sc))

            for j in range(N_DEV - 1):
                pltpu.make_async_remote_copy(
                    src_ref=ag_slots.at[j], dst_ref=ag_slots.at[j],
                    send_sem=ag_ssem.at[j], recv_sem=ag_rsem.at[j],
                    device_id=(my,),
                    device_id_type=pl.DeviceIdType.MESH).wait_recv()
                pltpu.make_async_remote_copy(
                    src_ref=ag_sc_slots.at[j], dst_ref=ag_sc_slots.at[j],
                    send_sem=sc_ssem.at[j], recv_sem=sc_rsem.at[j],
                    device_id=(my,),
                    device_id_type=pl.DeviceIdType.MESH).wait_recv()
                src = lax.rem(my + 1 + j, N_DEV)
                store_chunk(src, ag_slots[j].astype(jnp.float32)
                            * ag_sc_slots[j])

        if not _SKIP_RS:
            for c in range(N_DEV):
                @pl.when(my != c)
                def _(c=c):
                    for hp in range(HQ // 2):
                        rs_descriptor(c, hp).wait_send()
                    rs_lw_descriptor(c).wait_send()
        for rdma in ag_rdmas:
            rdma.wait_send()

        if not (_SKIP_RS and _SKIP_AG):
            @functools.partial(pl.run_scoped,
                               second_barrier=pltpu.SemaphoreType.REGULAR)
            def _(second_barrier):
                for o in range(1, N_DEV):
                    peer = lax.rem(my + o, N_DEV)
                    pl.semaphore_signal(second_barrier, inc=1,
                                        device_id=(peer,),
                                        device_id_type=pl.DeviceIdType.MESH)
                pl.semaphore_wait(second_barrier, N_DEV - 1)

    return pl.pallas_call(
        body,
        out_shape=jax.ShapeDtypeStruct((1, SQ, HQ * DH), jnp.float32),
        in_specs=[pl.BlockSpec(memory_space=pltpu.VMEM)] * 5,
        out_specs=pl.BlockSpec(memory_space=pltpu.VMEM),
        scratch_shapes=[
            pltpu.VMEM((HQ, SQ, DH), jnp.float32),
            pltpu.VMEM((SQ, 2 * HQ), jnp.float32),
            pltpu.VMEM((HQ, SQ, DH), jnp.int8),
            pltpu.VMEM((HQ // 2, N_DEV - 1, 2, CHUNK, DH),
                       jnp.int8),
            pltpu.VMEM((N_DEV - 1, CHUNK, 2 * HQ), jnp.float32),
            pltpu.VMEM((CHUNK, HQ * DH), jnp.int8),
            pltpu.VMEM((N_DEV - 1, CHUNK, HQ * DH), jnp.int8),
            pltpu.VMEM((CHUNK, 1), jnp.float32),
            pltpu.VMEM((N_DEV - 1, CHUNK, 1), jnp.float32),
            pltpu.SemaphoreType.DMA((HQ // 2, N_DEV - 1)),
            pltpu.SemaphoreType.DMA((HQ // 2, N_DEV - 1)),
            pltpu.SemaphoreType.DMA((N_DEV - 1,)),
            pltpu.SemaphoreType.DMA((N_DEV - 1,)),
            pltpu.SemaphoreType.DMA((N_DEV - 1,)),
            pltpu.SemaphoreType.DMA((N_DEV - 1,)),
            pltpu.SemaphoreType.DMA((N_DEV - 1,)),
            pltpu.SemaphoreType.DMA((N_DEV - 1,)),
        ],
        compiler_params=(None if (_SKIP_RS and _SKIP_AG)
                         else pltpu.CompilerParams(collective_id=0)),
    )(x, Wq, K_ext, V_ext, Wo)
